# baseline (speedup 1.0000x reference)
import jax
import jax.numpy as jnp
from jax.experimental import pallas as pl
from jax.experimental.pallas import tpu as pltpu

B, T, D = 4, 2048, 1024
H = D // 2
NUM_SLOTS = 64
TOP_K = 8
TBLK = 2048
NT = T // TBLK
NEG = -1e30


def _routing_dispatch_kernel(x_ref, w1a_ref, c1_ref, w2_ref, b2_ref,
                             den_ref, alpha_ref, si_ref):
    t = pl.program_id(1)
    xb = x_ref[0]
    h1t = jax.lax.dot_general(w1a_ref[...], xb, (((1,), (1,)), ((), ())),
                              preferred_element_type=jnp.float32)
    h1t = h1t + c1_ref[...]
    h1t = 0.5 * h1t * (1.0 + jax.lax.erf(h1t * 0.7071067811865476))
    lgt = jax.lax.dot_general(w2_ref[...], h1t, (((1,), (0,)), ((), ())),
                              preferred_element_type=jnp.float32)
    lgt = (lgt + b2_ref[...]) / den_ref[...]

    iota = jax.lax.broadcasted_iota(
        jnp.int32, (NUM_SLOTS, TBLK), 0).astype(jnp.float32)
    run = lgt
    v0 = jnp.max(run, axis=0, keepdims=True)
    alpha = jnp.zeros_like(lgt)
    denom = jnp.zeros((1, TBLK), jnp.float32)
    for _ in range(TOP_K):
        v = jnp.max(run, axis=0, keepdims=True)
        eq = run == v
        idx = jnp.min(jnp.where(eq, iota, 64.0), axis=0, keepdims=True)
        onehot = iota == idx
        w = jnp.exp(v - v0)
        alpha = jnp.where(onehot, jnp.broadcast_to(w, alpha.shape), alpha)
        denom = denom + w
        run = jnp.where(onehot, NEG, run)
    alpha = alpha / denom
    alpha_ref[0] = alpha.astype(jnp.bfloat16)

    contrib = jax.lax.dot_general(alpha, xb, (((1,), (0,)), ((), ())),
                                  preferred_element_type=jnp.float32)

    @pl.when(t == 0)
    def _():
        si_ref[0] = contrib

    @pl.when(t != 0)
    def _():
        si_ref[0] += contrib


def _nt(a, b):
    return jax.lax.dot_general(a, b, (((1,), (1,)), ((), ())),
                               preferred_element_type=jnp.float32)


def _gru_combine_kernel(alpha_blk_ref, alpha_full_ref, si_ref, hp0_ref,
                        wih_ref, whh_ref, bih_ref, bhh_ref, wv_ref, wo_ref,
                        bv_ref, bo_ref, out_ref, svo_s):
    b = pl.program_id(0)
    t = pl.program_id(1)

    @pl.when((b == 0) & (t == 0))
    def _():
        ws = jnp.sum(alpha_full_ref[...].astype(jnp.float32), axis=2) + 1e-8
        xi = (si_ref[...] / ws[..., None]).reshape(B * NUM_SLOTS, D)
        gi = _nt(xi, wih_ref[...]) + bih_ref[...]
        hp0 = hp0_ref[...]
        gh0 = _nt(hp0, whh_ref[...]) + bhh_ref[...]
        gh = jnp.broadcast_to(gh0[None], (B, NUM_SLOTS, 3 * D))
        gh = gh.reshape(B * NUM_SLOTS, 3 * D)
        hp = jnp.broadcast_to(hp0[None], (B, NUM_SLOTS, D))
        hp = hp.reshape(B * NUM_SLOTS, D)
        r = jax.nn.sigmoid(gi[:, :D] + gh[:, :D])
        z = jax.nn.sigmoid(gi[:, D:2 * D] + gh[:, D:2 * D])
        n = jnp.tanh(gi[:, 2 * D:] + r * gh[:, 2 * D:])
        hnew = (1.0 - z) * n + z * hp
        sv = _nt(hnew, wv_ref[...]) + bv_ref[...]
        svo = _nt(sv, wo_ref[...]).reshape(B, NUM_SLOTS, D)
        svo_s[...] = svo.astype(jnp.bfloat16)

    out = jax.lax.dot_general(alpha_blk_ref[0], svo_s[b],
                              (((0,), (0,)), ((), ())),
                              preferred_element_type=jnp.float32)
    out_ref[0] = out + bo_ref[...]


@jax.jit
def kernel(x, slot_init, slot_scale, W1, b1, W2, b2, W_ih, W_hh, b_ih, b_hh,
           Wv, bv, Wo, bo, tau):
    f32 = jnp.float32
    bf16 = jnp.bfloat16
    hp0 = slot_init * slot_scale
    slot_mean = jnp.mean(hp0, axis=0)
    c1 = (b1 + W1[:, D:] @ slot_mean).reshape(H, 1)
    den = (jnp.abs(tau) + 0.1).reshape(1, 1)

    alpha_t, slot_input = pl.pallas_call(
        _routing_dispatch_kernel,
        grid=(B, NT),
        in_specs=[
            pl.BlockSpec((1, TBLK, D), lambda b, t: (b, t, 0)),
            pl.BlockSpec((H, D), lambda b, t: (0, 0)),
            pl.BlockSpec((H, 1), lambda b, t: (0, 0)),
            pl.BlockSpec((NUM_SLOTS, H), lambda b, t: (0, 0)),
            pl.BlockSpec((NUM_SLOTS, 1), lambda b, t: (0, 0)),
            pl.BlockSpec((1, 1), lambda b, t: (0, 0)),
        ],
        out_specs=[
            pl.BlockSpec((1, NUM_SLOTS, TBLK), lambda b, t: (b, 0, t)),
            pl.BlockSpec((1, NUM_SLOTS, D), lambda b, t: (b, 0, 0)),
        ],
        out_shape=[
            jax.ShapeDtypeStruct((B, NUM_SLOTS, T), bf16),
            jax.ShapeDtypeStruct((B, NUM_SLOTS, D), f32),
        ],
    )(x, W1[:, :D], c1, W2, b2.reshape(NUM_SLOTS, 1), den)

    out = pl.pallas_call(
        _gru_combine_kernel,
        grid=(B, NT),
        in_specs=[
            pl.BlockSpec((1, NUM_SLOTS, TBLK), lambda b, t: (b, 0, t)),
            pl.BlockSpec((B, NUM_SLOTS, T), lambda b, t: (0, 0, 0)),
            pl.BlockSpec((B, NUM_SLOTS, D), lambda b, t: (0, 0, 0)),
            pl.BlockSpec((NUM_SLOTS, D), lambda b, t: (0, 0)),
            pl.BlockSpec((3 * D, D), lambda b, t: (0, 0)),
            pl.BlockSpec((3 * D, D), lambda b, t: (0, 0)),
            pl.BlockSpec((1, 3 * D), lambda b, t: (0, 0)),
            pl.BlockSpec((1, 3 * D), lambda b, t: (0, 0)),
            pl.BlockSpec((D, D), lambda b, t: (0, 0)),
            pl.BlockSpec((D, D), lambda b, t: (0, 0)),
            pl.BlockSpec((1, D), lambda b, t: (0, 0)),
            pl.BlockSpec((1, D), lambda b, t: (0, 0)),
        ],
        out_specs=pl.BlockSpec((1, TBLK, D), lambda b, t: (b, t, 0)),
        out_shape=jax.ShapeDtypeStruct((B, T, D), f32),
        scratch_shapes=[pltpu.VMEM((B, NUM_SLOTS, D), bf16)],
    )(alpha_t, alpha_t, slot_input, hp0, W_ih, W_hh,
      b_ih.reshape(1, 3 * D), b_hh.reshape(1, 3 * D), Wv, Wo,
      bv.reshape(1, D), bo.reshape(1, D))
    return out
